# baseline (speedup 1.0000x reference)
import functools
import jax
import jax.numpy as jnp
from jax import lax
from jax.experimental import pallas as pl
from jax.experimental.pallas import tpu as pltpu
from jax.experimental.pallas import tpu_sc as plsc

_L = 16
_C = 64
_NBUF = 2


def _sc_body(tot, D, nw, x_hbm, pos_hbm, tab_hbm, out_hbm,
             ibufs, gbufs, idxs, in_sems, g_sems, out_sems, idx_sems):
    rows_per_w = tot // nw
    nchunks = rows_per_w // _C
    wid = lax.axis_index("s") * 2 + lax.axis_index("c")
    row0 = wid * rows_per_w

    def start_in(g, slot):
        base = row0 + g * _C
        pltpu.make_async_copy(
            x_hbm.at[pl.ds(base, _C)], ibufs.at[slot], in_sems.at[slot]
        ).start()
        pltpu.make_async_copy(
            pos_hbm.at[pl.ds(row0 + g * _C, _C)], idxs.at[slot],
            idx_sems.at[slot],
        ).start()

    def wait_in(slot):
        pltpu.make_async_copy(
            x_hbm.at[pl.ds(0, _C)], ibufs.at[slot], in_sems.at[slot]
        ).wait()

    def start_gather(slot):
        pltpu.make_async_copy(
            pos_hbm.at[pl.ds(0, _C)], idxs.at[slot], idx_sems.at[slot]
        ).wait()
        pltpu.make_async_copy(
            tab_hbm.at[idxs.at[slot]], gbufs.at[slot], g_sems.at[slot]
        ).start()

    def wait_gather(slot):
        pltpu.make_async_copy(
            tab_hbm.at[idxs.at[slot]], gbufs.at[slot], g_sems.at[slot]
        ).wait()

    def start_out(g, slot):
        base = row0 + g * _C
        pltpu.make_async_copy(
            gbufs.at[slot], out_hbm.at[pl.ds(base, _C)], out_sems.at[slot]
        ).start()

    def wait_out(slot):
        pltpu.make_async_copy(
            gbufs.at[slot], out_hbm.at[pl.ds(0, _C)], out_sems.at[slot]
        ).wait()

    start_in(0, 0)
    start_gather(0)

    def chunk_step(g, _):
        slot = lax.rem(g, _NBUF)
        nslot = lax.rem(g + 1, _NBUF)

        @pl.when(g + 1 < nchunks)
        def _():
            @pl.when(g + 1 >= _NBUF)
            def _():
                wait_out(nslot)
            start_in(g + 1, nslot)
            start_gather(nslot)

        wait_in(slot)
        wait_gather(slot)

        @plsc.parallel_loop(0, _C, 1, unroll=2)
        def row_step(r):
            for c in range(D // _L):
                x = ibufs[slot, r, pl.ds(c * _L, _L)]
                t = gbufs[slot, r, pl.ds(c * _L, _L)]
                gbufs[slot, r, pl.ds(c * _L, _L)] = x + t

        start_out(g, slot)
        return 0

    lax.fori_loop(0, nchunks, chunk_step, 0, unroll=False)
    for s in range(_NBUF):
        wait_out(s)


def kernel(inputs, inputs_positions, position_emb):
    B, N, D = inputs.shape
    tot = B * N
    info = plsc.get_sparse_core_info()
    nw = info.num_cores * info.num_subcores

    x = inputs.reshape(tot, D)
    pos = inputs_positions.reshape(tot).astype(jnp.int32)
    table = jnp.squeeze(position_emb, axis=0)

    mesh = plsc.VectorSubcoreMesh(core_axis_name="c", subcore_axis_name="s")
    out = pl.kernel(
        functools.partial(_sc_body, tot, D, nw),
        out_type=jax.ShapeDtypeStruct((tot, D), jnp.float32),
        mesh=mesh,
        scratch_types=[
            pltpu.VMEM((_NBUF, _C, D), jnp.float32),
            pltpu.VMEM((_NBUF, _C, D), jnp.float32),
            pltpu.VMEM((_NBUF, _C), jnp.int32),
            pltpu.SemaphoreType.DMA((_NBUF,)),
            pltpu.SemaphoreType.DMA((_NBUF,)),
            pltpu.SemaphoreType.DMA((_NBUF,)),
            pltpu.SemaphoreType.DMA((_NBUF,)),
        ],
    )(x, pos, table)
    return out.reshape(B, N, D)
